# baseline (device time: 49345 ns/iter reference)
import jax
import jax.numpy as jnp
from jax import lax
from jax.experimental import pallas as pl
from jax.experimental.pallas import tpu as pltpu

N_DEV = 16
SQ = 512
D = 1024
HQ_LOCAL = 8
DH = 128
GROUP = 4
KV_COLS = 2 * DH
CHUNK = SQ // N_DEV
BLK = 128
SCALE = 0.08838834764831843


def kernel(x, Wq, Wo, Wk, Wv):
    def body(x_ref, wq_ref, wo_ref, wk_hbm, wv_hbm, out_ref,
             wk_ref, wv_ref, send_ref, a2a_ref,
             copy_sems, pa_send, pa_recv, pb_send, pb_recv):
        my = lax.axis_index("i")

        ck = pltpu.make_async_copy(
            wk_hbm.at[:, pl.ds(my * KV_COLS, KV_COLS)], wk_ref,
            copy_sems.at[0])
        cv = pltpu.make_async_copy(
            wv_hbm.at[:, pl.ds(my * KV_COLS, KV_COLS)], wv_ref,
            copy_sems.at[1])
        ck.start()
        cv.start()

        barrier_sem = pltpu.get_barrier_semaphore()
        for d in range(1, N_DEV):
            tgt = lax.rem(my + d, N_DEV)
            pl.semaphore_signal(
                barrier_sem, inc=1,
                device_id=(tgt,), device_id_type=pl.DeviceIdType.MESH,
            )
        pl.semaphore_wait(barrier_sem, N_DEV - 1)
        ck.wait()
        cv.wait()

        xb = x_ref[0].astype(jnp.bfloat16)
        q = jnp.dot(xb, wq_ref[...].astype(jnp.bfloat16),
                    preferred_element_type=jnp.float32) * SCALE
        k = jnp.dot(xb, wk_ref[...].astype(jnp.bfloat16),
                    preferred_element_type=jnp.float32)
        v = jnp.dot(xb, wv_ref[...].astype(jnp.bfloat16),
                    preferred_element_type=jnp.float32)
        kb = k.astype(jnp.bfloat16)
        vb = v.astype(jnp.bfloat16)
        wob = wo_ref[...].astype(jnp.bfloat16)
        pa = []
        n_blk = SQ // BLK
        for blk in range(n_blk):
            r0 = blk * BLK
            pblk = jnp.zeros((BLK, D), jnp.float32)
            for h in range(HQ_LOCAL):
                kv = h // GROUP
                qh = q[r0:r0 + BLK, h * DH:(h + 1) * DH].astype(jnp.bfloat16)
                kh = kb[:, kv * DH:(kv + 1) * DH]
                vh = vb[:, kv * DH:(kv + 1) * DH]
                p = jnp.exp(lax.dot_general(
                    qh, kh, (((1,), (1,)), ((), ())),
                    preferred_element_type=jnp.float32))
                l_inv = 1.0 / jnp.sum(p, axis=1, keepdims=True)
                o = jnp.dot(p.astype(jnp.bfloat16), vh,
                            preferred_element_type=jnp.float32) * l_inv
                pblk = pblk + jnp.dot(
                    o.astype(jnp.bfloat16),
                    wob[h * DH:(h + 1) * DH, :],
                    preferred_element_type=jnp.float32)
            cpb = BLK // CHUNK
            send_ref[pl.ds(blk * cpb, cpb)] = pblk.astype(
                jnp.bfloat16).reshape(cpb, CHUNK, D)
            for j in range(cpb):
                c = blk * cpb + j
                rdma = pltpu.make_async_remote_copy(
                    src_ref=send_ref.at[c],
                    dst_ref=a2a_ref.at[my],
                    send_sem=pa_send.at[c],
                    recv_sem=pa_recv.at[my],
                    device_id=(c,),
                    device_id_type=pl.DeviceIdType.MESH,
                )

                @pl.when(my != c)
                def _(rdma=rdma):
                    rdma.start()

                @pl.when(my == c)
                def _(c=c):
                    a2a_ref[my] = send_ref[c]

                pa.append((c, rdma))
        for d in range(1, N_DEV):
            src = lax.rem(my + d, N_DEV)
            pltpu.make_async_remote_copy(
                src_ref=send_ref.at[src],
                dst_ref=a2a_ref.at[src],
                send_sem=pa_send.at[src],
                recv_sem=pa_recv.at[src],
                device_id=(src,),
                device_id_type=pl.DeviceIdType.MESH,
            ).wait_recv()

        red = jnp.sum(a2a_ref[...].astype(jnp.float32), axis=0)
        out_ref[my] = red.astype(jnp.bfloat16)
        pb = []
        for d in range(1, N_DEV):
            tgt = lax.rem(my + d, N_DEV)
            rdma = pltpu.make_async_remote_copy(
                src_ref=out_ref.at[my],
                dst_ref=out_ref.at[my],
                send_sem=pb_send.at[tgt],
                recv_sem=pb_recv.at[my],
                device_id=(tgt,),
                device_id_type=pl.DeviceIdType.MESH,
            )
            rdma.start()
            pb.append(rdma)
        for c, r in pa:
            @pl.when(my != c)
            def _(r=r):
                r.wait_send()
        for d in range(1, N_DEV):
            src = lax.rem(my + d, N_DEV)
            pltpu.make_async_remote_copy(
                src_ref=out_ref.at[src],
                dst_ref=out_ref.at[src],
                send_sem=pb_send.at[src],
                recv_sem=pb_recv.at[src],
                device_id=(src,),
                device_id_type=pl.DeviceIdType.MESH,
            ).wait_recv()
        for r in pb:
            r.wait_send()

    out = pl.pallas_call(
        body,
        out_shape=jax.ShapeDtypeStruct((N_DEV, CHUNK, D), jnp.bfloat16),
        in_specs=[pl.BlockSpec(memory_space=pltpu.VMEM)] * 3
        + [pl.BlockSpec(memory_space=pl.ANY)] * 2,
        out_specs=pl.BlockSpec(memory_space=pltpu.VMEM),
        scratch_shapes=[
            pltpu.VMEM((D, KV_COLS), jnp.float32),
            pltpu.VMEM((D, KV_COLS), jnp.float32),
            pltpu.VMEM((N_DEV, CHUNK, D), jnp.bfloat16),
            pltpu.VMEM((N_DEV, CHUNK, D), jnp.bfloat16),
            pltpu.SemaphoreType.DMA((2,)),
            pltpu.SemaphoreType.DMA((N_DEV,)),
            pltpu.SemaphoreType.DMA((N_DEV,)),
            pltpu.SemaphoreType.DMA((N_DEV,)),
            pltpu.SemaphoreType.DMA((N_DEV,)),
        ],
        compiler_params=pltpu.CompilerParams(collective_id=0),
    )(x, Wq, Wo, Wk, Wv)
    return out.reshape(1, SQ, D)


# device time: 43507 ns/iter; 1.1342x vs baseline; 1.1342x over previous
import jax
import jax.numpy as jnp
from jax import lax
from jax.experimental import pallas as pl
from jax.experimental.pallas import tpu as pltpu

N_DEV = 16
SQ = 512
D = 1024
HQ_LOCAL = 8
DH = 128
GROUP = 4
KV_COLS = 2 * DH
CHUNK = SQ // N_DEV
BLK = 256
SCALE = 0.08838834764831843


def kernel(x, Wq, Wo, Wk, Wv):
    def body(x_ref, wq_ref, wo_ref, wk_hbm, wv_hbm, out_ref,
             wk_ref, wv_ref, send_ref, a2a_ref,
             copy_sems, pa_send, pa_recv, pb_send, pb_recv):
        my = lax.axis_index("i")

        ck = pltpu.make_async_copy(
            wk_hbm.at[:, pl.ds(my * KV_COLS, KV_COLS)], wk_ref,
            copy_sems.at[0])
        cv = pltpu.make_async_copy(
            wv_hbm.at[:, pl.ds(my * KV_COLS, KV_COLS)], wv_ref,
            copy_sems.at[1])
        ck.start()
        cv.start()

        barrier_sem = pltpu.get_barrier_semaphore()
        for d in range(1, N_DEV):
            tgt = lax.rem(my + d, N_DEV)
            pl.semaphore_signal(
                barrier_sem, inc=1,
                device_id=(tgt,), device_id_type=pl.DeviceIdType.MESH,
            )
        ck.wait()
        cv.wait()

        xb = x_ref[0].astype(jnp.bfloat16)
        q = jnp.dot(xb, wq_ref[...].astype(jnp.bfloat16),
                    preferred_element_type=jnp.float32) * SCALE
        kb = jnp.dot(xb, wk_ref[...].astype(jnp.bfloat16),
                     preferred_element_type=jnp.float32).astype(jnp.bfloat16)
        vb = jnp.dot(xb, wv_ref[...].astype(jnp.bfloat16),
                     preferred_element_type=jnp.float32).astype(jnp.bfloat16)
        wob = wo_ref[...].astype(jnp.bfloat16)

        pa = []
        cpb = BLK // CHUNK
        for blk in range(SQ // BLK):
            r0 = blk * BLK
            pblk = jnp.zeros((BLK, D), jnp.float32)
            for h in range(HQ_LOCAL):
                kv = h // GROUP
                qh = q[r0:r0 + BLK, h * DH:(h + 1) * DH].astype(jnp.bfloat16)
                kh = kb[:, kv * DH:(kv + 1) * DH]
                vh = vb[:, kv * DH:(kv + 1) * DH]
                p = jnp.exp(lax.dot_general(
                    qh, kh, (((1,), (1,)), ((), ())),
                    preferred_element_type=jnp.float32))
                l_inv = 1.0 / jnp.sum(p, axis=1, keepdims=True)
                o = jnp.dot(p.astype(jnp.bfloat16), vh,
                            preferred_element_type=jnp.float32) * l_inv
                pblk = pblk + jnp.dot(
                    o.astype(jnp.bfloat16),
                    wob[h * DH:(h + 1) * DH, :],
                    preferred_element_type=jnp.float32)
            send_ref[pl.ds(blk * cpb, cpb)] = pblk.astype(
                jnp.bfloat16).reshape(cpb, CHUNK, D)
            if blk == 0:
                pl.semaphore_wait(barrier_sem, N_DEV - 1)
            for j in range(cpb):
                c = blk * cpb + j
                rdma = pltpu.make_async_remote_copy(
                    src_ref=send_ref.at[c],
                    dst_ref=a2a_ref.at[my],
                    send_sem=pa_send.at[c],
                    recv_sem=pa_recv.at[my],
                    device_id=(c,),
                    device_id_type=pl.DeviceIdType.MESH,
                )

                @pl.when(my != c)
                def _(rdma=rdma):
                    rdma.start()

                pa.append((c, rdma))
        a2a_ref[my] = send_ref[my]

        for d in range(1, N_DEV):
            src = lax.rem(my + d, N_DEV)
            pltpu.make_async_remote_copy(
                src_ref=send_ref.at[src],
                dst_ref=a2a_ref.at[src],
                send_sem=pa_send.at[src],
                recv_sem=pa_recv.at[src],
                device_id=(src,),
                device_id_type=pl.DeviceIdType.MESH,
            ).wait_recv()

        red = jnp.sum(a2a_ref[...].astype(jnp.float32), axis=0)
        out_ref[my] = red.astype(jnp.bfloat16)
        pb = []
        for d in range(1, N_DEV):
            tgt = lax.rem(my + d, N_DEV)
            rdma = pltpu.make_async_remote_copy(
                src_ref=out_ref.at[my],
                dst_ref=out_ref.at[my],
                send_sem=pb_send.at[tgt],
                recv_sem=pb_recv.at[my],
                device_id=(tgt,),
                device_id_type=pl.DeviceIdType.MESH,
            )
            rdma.start()
            pb.append(rdma)
        for c, r in pa:
            @pl.when(my != c)
            def _(r=r):
                r.wait_send()
        for d in range(1, N_DEV):
            src = lax.rem(my + d, N_DEV)
            pltpu.make_async_remote_copy(
                src_ref=out_ref.at[src],
                dst_ref=out_ref.at[src],
                send_sem=pb_send.at[src],
                recv_sem=pb_recv.at[src],
                device_id=(src,),
                device_id_type=pl.DeviceIdType.MESH,
            ).wait_recv()
        for r in pb:
            r.wait_send()

    out = pl.pallas_call(
        body,
        out_shape=jax.ShapeDtypeStruct((N_DEV, CHUNK, D), jnp.bfloat16),
        in_specs=[pl.BlockSpec(memory_space=pltpu.VMEM)] * 3
        + [pl.BlockSpec(memory_space=pl.ANY)] * 2,
        out_specs=pl.BlockSpec(memory_space=pltpu.VMEM),
        scratch_shapes=[
            pltpu.VMEM((D, KV_COLS), jnp.float32),
            pltpu.VMEM((D, KV_COLS), jnp.float32),
            pltpu.VMEM((N_DEV, CHUNK, D), jnp.bfloat16),
            pltpu.VMEM((N_DEV, CHUNK, D), jnp.bfloat16),
            pltpu.SemaphoreType.DMA((2,)),
            pltpu.SemaphoreType.DMA((N_DEV,)),
            pltpu.SemaphoreType.DMA((N_DEV,)),
            pltpu.SemaphoreType.DMA((N_DEV,)),
            pltpu.SemaphoreType.DMA((N_DEV,)),
        ],
        compiler_params=pltpu.CompilerParams(collective_id=0),
    )(x, Wq, Wo, Wk, Wv)
    return out.reshape(1, SQ, D)


# device time: 40250 ns/iter; 1.2260x vs baseline; 1.0809x over previous
import jax
import jax.numpy as jnp
from jax import lax
from jax.experimental import pallas as pl
from jax.experimental.pallas import tpu as pltpu

N_DEV = 16
SQ = 512
D = 1024
HQ_LOCAL = 8
DH = 128
GROUP = 4
KV_LOCAL = 2
KV_COLS = KV_LOCAL * DH
CHUNK = SQ // N_DEV
SCALE = 0.08838834764831843


def kernel(x, Wq, Wo, Wk, Wv):
    def body(x_ref, wq_ref, wo_ref, wk_hbm, wv_hbm, out_ref,
             wk_ref, wv_ref, send_ref, a2a_ref,
             copy_sems, pa_send, pa_recv, pb_send, pb_recv):
        my = lax.axis_index("i")

        ck = pltpu.make_async_copy(
            wk_hbm.at[:, pl.ds(my * KV_COLS, KV_COLS)], wk_ref,
            copy_sems.at[0])
        cv = pltpu.make_async_copy(
            wv_hbm.at[:, pl.ds(my * KV_COLS, KV_COLS)], wv_ref,
            copy_sems.at[1])
        ck.start()
        cv.start()

        barrier_sem = pltpu.get_barrier_semaphore()
        for d in range(1, N_DEV):
            tgt = lax.rem(my + d, N_DEV)
            pl.semaphore_signal(
                barrier_sem, inc=1,
                device_id=(tgt,), device_id_type=pl.DeviceIdType.MESH,
            )
        ck.wait()
        cv.wait()

        xb = x_ref[0].astype(jnp.bfloat16)
        q = jnp.dot(xb, wq_ref[...].astype(jnp.bfloat16),
                    preferred_element_type=jnp.float32) * SCALE
        kb = jnp.dot(xb, wk_ref[...].astype(jnp.bfloat16),
                     preferred_element_type=jnp.float32).astype(jnp.bfloat16)
        vb = jnp.dot(xb, wv_ref[...].astype(jnp.bfloat16),
                     preferred_element_type=jnp.float32).astype(jnp.bfloat16)

        attn_parts = []
        for g in range(KV_LOCAL):
            qg = jnp.concatenate(
                [q[:, (g * GROUP + j) * DH:(g * GROUP + j + 1) * DH]
                 for j in range(GROUP)], axis=0).astype(jnp.bfloat16)
            kh = kb[:, g * DH:(g + 1) * DH]
            vh = vb[:, g * DH:(g + 1) * DH]
            p = jnp.exp(lax.dot_general(
                qg, kh, (((1,), (1,)), ((), ())),
                preferred_element_type=jnp.float32))
            l_inv = 1.0 / jnp.sum(p, axis=1, keepdims=True)
            og = jnp.dot(p.astype(jnp.bfloat16), vh,
                         preferred_element_type=jnp.float32) * l_inv
            attn_parts += [og[j * SQ:(j + 1) * SQ].astype(jnp.bfloat16)
                           for j in range(GROUP)]
        attn = jnp.concatenate(attn_parts, axis=1)
        partial = jnp.dot(attn, wo_ref[...].astype(jnp.bfloat16),
                          preferred_element_type=jnp.float32)

        send_ref[...] = partial.astype(jnp.bfloat16).reshape(N_DEV, CHUNK, D)
        a2a_ref[my] = send_ref[my]
        pl.semaphore_wait(barrier_sem, N_DEV - 1)
        pa = []
        for d in range(1, N_DEV):
            tgt = lax.rem(my + d, N_DEV)
            rdma = pltpu.make_async_remote_copy(
                src_ref=send_ref.at[tgt],
                dst_ref=a2a_ref.at[my],
                send_sem=pa_send.at[tgt],
                recv_sem=pa_recv.at[my],
                device_id=(tgt,),
                device_id_type=pl.DeviceIdType.MESH,
            )
            rdma.start()
            pa.append(rdma)
        for d in range(1, N_DEV):
            src = lax.rem(my + d, N_DEV)
            pltpu.make_async_remote_copy(
                src_ref=send_ref.at[src],
                dst_ref=a2a_ref.at[src],
                send_sem=pa_send.at[src],
                recv_sem=pa_recv.at[src],
                device_id=(src,),
                device_id_type=pl.DeviceIdType.MESH,
            ).wait_recv()

        red = jnp.sum(a2a_ref[...].astype(jnp.float32), axis=0)
        out_ref[my] = red.astype(jnp.bfloat16)
        pb = []
        for d in range(1, N_DEV):
            tgt = lax.rem(my + d, N_DEV)
            rdma = pltpu.make_async_remote_copy(
                src_ref=out_ref.at[my],
                dst_ref=out_ref.at[my],
                send_sem=pb_send.at[tgt],
                recv_sem=pb_recv.at[my],
                device_id=(tgt,),
                device_id_type=pl.DeviceIdType.MESH,
            )
            rdma.start()
            pb.append(rdma)
        for r in pa:
            r.wait_send()
        for d in range(1, N_DEV):
            src = lax.rem(my + d, N_DEV)
            pltpu.make_async_remote_copy(
                src_ref=out_ref.at[src],
                dst_ref=out_ref.at[src],
                send_sem=pb_send.at[src],
                recv_sem=pb_recv.at[src],
                device_id=(src,),
                device_id_type=pl.DeviceIdType.MESH,
            ).wait_recv()
        for r in pb:
            r.wait_send()

    out = pl.pallas_call(
        body,
        out_shape=jax.ShapeDtypeStruct((N_DEV, CHUNK, D), jnp.bfloat16),
        in_specs=[pl.BlockSpec(memory_space=pltpu.VMEM)] * 3
        + [pl.BlockSpec(memory_space=pl.ANY)] * 2,
        out_specs=pl.BlockSpec(memory_space=pltpu.VMEM),
        scratch_shapes=[
            pltpu.VMEM((D, KV_COLS), jnp.float32),
            pltpu.VMEM((D, KV_COLS), jnp.float32),
            pltpu.VMEM((N_DEV, CHUNK, D), jnp.bfloat16),
            pltpu.VMEM((N_DEV, CHUNK, D), jnp.bfloat16),
            pltpu.SemaphoreType.DMA((2,)),
            pltpu.SemaphoreType.DMA((N_DEV,)),
            pltpu.SemaphoreType.DMA((N_DEV,)),
            pltpu.SemaphoreType.DMA((N_DEV,)),
            pltpu.SemaphoreType.DMA((N_DEV,)),
        ],
        compiler_params=pltpu.CompilerParams(collective_id=0),
    )(x, Wq, Wo, Wk, Wv)
    return out.reshape(1, SQ, D)
